# baseline (speedup 1.0000x reference)
import functools

import jax
import jax.numpy as jnp
from jax import lax
from jax.experimental import pallas as pl
from jax.experimental.pallas import tpu as pltpu
from jax.experimental.pallas import tpu_sc as plsc

D_ITEM = 64
D_USER = 64
HIDDEN = 128
T = 200
B = 1024
K = 10
ITERS = 10

NW = 32
ROWS_W = (B * T) // NW
GCH = 128
NCH_G = 10
NSUP = ROWS_W // (GCH * NCH_G)
U_W = B // NW


def _sc_gather_body(seq_hbm, uid_hbm, item_hbm, user_hbm, s_out, u_out,
                    idx_v, rows_v, uidx_v, urows_v, sem):
    wid = lax.axis_index("s") * 2 + lax.axis_index("c")
    pltpu.sync_copy(uid_hbm.at[pl.ds(wid * U_W, U_W)], uidx_v)
    pltpu.async_copy(user_hbm.at[uidx_v], urows_v, sem).wait()
    pltpu.sync_copy(urows_v, u_out.at[pl.ds(wid * U_W, U_W)])
    pltpu.sync_copy(seq_hbm.at[wid], idx_v)

    def sup(s, _):
        for j in range(NCH_G):
            pltpu.async_copy(item_hbm.at[idx_v.at[s * NCH_G + j]],
                             rows_v.at[j], sem).wait()
        pltpu.sync_copy(
            rows_v,
            s_out.at[pl.ds(wid * (ROWS_W // GCH) + s * NCH_G, NCH_G)])
        return ()

    lax.fori_loop(0, NSUP, sup, (), unroll=False)


@jax.jit
def _sc_gather(seq3d, user_ids, item_table, user_table):
    mesh = plsc.VectorSubcoreMesh(core_axis_name="c", subcore_axis_name="s")
    f = pl.kernel(
        _sc_gather_body,
        mesh=mesh,
        out_type=[
            jax.ShapeDtypeStruct((B * T // GCH, GCH, D_ITEM), jnp.float32),
            jax.ShapeDtypeStruct((B, D_USER), jnp.float32),
        ],
        scratch_types=[
            pltpu.VMEM((ROWS_W // GCH, GCH), jnp.int32),
            pltpu.VMEM((NCH_G, GCH, D_ITEM), jnp.float32),
            pltpu.VMEM((U_W,), jnp.int32),
            pltpu.VMEM((U_W, D_USER), jnp.float32),
            pltpu.SemaphoreType.DMA,
        ],
        compiler_params=pltpu.CompilerParams(use_tc_tiling_on_sc=False),
    )
    return f(seq3d, user_ids, item_table, user_table)


BB = 8
KP = 16
GK = BB * KP
N = BB * T
PADC = 1e15
_DOT = dict(preferred_element_type=jnp.float32,
            precision=jax.lax.Precision.HIGHEST)


def _kmeans_body(s_ref, u_ref, pos_ref, h_ref, ids_ref, cen_ref):
    s = s_ref[...]
    u = u_ref[...]
    pos = pos_ref[...]
    ub = jnp.broadcast_to(u[:, None, :], (BB, T, D_USER))
    h = jnp.concatenate([s, ub], axis=2) + pos[None, :, :]
    h_ref[...] = h
    XT3 = [h[b].T for b in range(BB)]

    kiota = lax.broadcasted_iota(jnp.int32, (BB, KP, T), 1)
    crow = lax.broadcasted_iota(jnp.int32, (BB, KP, HIDDEN), 1)

    C0 = jnp.where(crow < K, h[:, :KP, :], PADC)

    def assign(C):
        G = jnp.stack([
            lax.dot_general(C[b], XT3[b], (((1,), (0,)), ((), ())), **_DOT)
            for b in range(BB)])
        c2 = jnp.sum(C * C, axis=2, keepdims=True)
        d3 = c2 - 2.0 * G
        dmin = jnp.min(d3, axis=1, keepdims=True)
        return jnp.min(jnp.where(d3 == dmin, kiota, KP), axis=1)

    def body(i, C):
        ids = assign(C)
        O = (kiota == ids[:, None, :]).astype(jnp.float32)
        sums = jnp.stack([
            lax.dot_general(O[b], h[b], (((1,), (0,)), ((), ())), **_DOT)
            for b in range(BB)])
        cnt = jnp.sum(O, axis=2, keepdims=True)
        return jnp.where(cnt > 0.0, sums / jnp.maximum(cnt, 1.0), C)

    C = lax.fori_loop(0, ITERS, body, C0)
    ids_ref[...] = assign(C)
    cen_ref[...] = C


@jax.jit
def _kmeans_tc(s, u, pos):
    return pl.pallas_call(
        _kmeans_body,
        grid=(B // BB,),
        in_specs=[
            pl.BlockSpec((BB, T, D_ITEM), lambda i: (i, 0, 0)),
            pl.BlockSpec((BB, D_USER), lambda i: (i, 0)),
            pl.BlockSpec((T, HIDDEN), lambda i: (0, 0)),
        ],
        out_specs=[
            pl.BlockSpec((BB, T, HIDDEN), lambda i: (i, 0, 0)),
            pl.BlockSpec((BB, T), lambda i: (i, 0)),
            pl.BlockSpec((BB, KP, HIDDEN), lambda i: (i, 0, 0)),
        ],
        out_shape=[
            jax.ShapeDtypeStruct((B, T, HIDDEN), jnp.float32),
            jax.ShapeDtypeStruct((B, T), jnp.int32),
            jax.ShapeDtypeStruct((B, KP, HIDDEN), jnp.float32),
        ],
    )(s, u, pos)


def kernel(user_ids, seq, pos_seqs, neg_seqs, item_table, user_table, pos_table):
    seq3d = seq.astype(jnp.int32).reshape(NW, ROWS_W // GCH, GCH)
    uids = user_ids.astype(jnp.int32)
    s3d, u = _sc_gather(seq3d, uids, item_table, user_table)
    s = s3d.reshape(B, T, D_ITEM)
    h, ids, cen_r = _kmeans_tc(s, u, pos_table)
    return h, ids, cen_r[:, :K, :]


# speedup vs baseline: 1.0716x; 1.0716x over previous
import functools

import jax
import jax.numpy as jnp
from jax import lax
from jax.experimental import pallas as pl
from jax.experimental.pallas import tpu as pltpu
from jax.experimental.pallas import tpu_sc as plsc

D_ITEM = 64
D_USER = 64
HIDDEN = 128
T = 200
B = 1024
K = 10
ITERS = 10

NW = 32
ROWS_W = (B * T) // NW
GCH = 128
NCH_G = 10
NSUP = ROWS_W // (GCH * NCH_G)
U_W = B // NW


def _sc_gather_body(seq_hbm, uid_hbm, item_hbm, user_hbm, s_out, u_out,
                    idx_v, rows_v, uidx_v, urows_v, sem):
    wid = lax.axis_index("s") * 2 + lax.axis_index("c")
    pltpu.sync_copy(uid_hbm.at[pl.ds(wid * U_W, U_W)], uidx_v)
    pltpu.async_copy(user_hbm.at[uidx_v], urows_v, sem).wait()
    pltpu.sync_copy(urows_v, u_out.at[pl.ds(wid * U_W, U_W)])
    pltpu.sync_copy(seq_hbm.at[wid], idx_v)

    def sup(s, _):
        for j in range(NCH_G):
            pltpu.async_copy(item_hbm.at[idx_v.at[s * NCH_G + j]],
                             rows_v.at[j], sem).wait()
        pltpu.sync_copy(
            rows_v,
            s_out.at[pl.ds(wid * (ROWS_W // GCH) + s * NCH_G, NCH_G)])
        return ()

    lax.fori_loop(0, NSUP, sup, (), unroll=False)


@jax.jit
def _sc_gather(seq3d, user_ids, item_table, user_table):
    mesh = plsc.VectorSubcoreMesh(core_axis_name="c", subcore_axis_name="s")
    f = pl.kernel(
        _sc_gather_body,
        mesh=mesh,
        out_type=[
            jax.ShapeDtypeStruct((B * T // GCH, GCH, D_ITEM), jnp.float32),
            jax.ShapeDtypeStruct((B, D_USER), jnp.float32),
        ],
        scratch_types=[
            pltpu.VMEM((ROWS_W // GCH, GCH), jnp.int32),
            pltpu.VMEM((NCH_G, GCH, D_ITEM), jnp.float32),
            pltpu.VMEM((U_W,), jnp.int32),
            pltpu.VMEM((U_W, D_USER), jnp.float32),
            pltpu.SemaphoreType.DMA,
        ],
        compiler_params=pltpu.CompilerParams(use_tc_tiling_on_sc=False),
    )
    return f(seq3d, user_ids, item_table, user_table)


BB = 4
KP = 16
GK = BB * KP
N = BB * T
PADC = 1e15
_DOT = dict(preferred_element_type=jnp.float32,
            precision=jax.lax.Precision.HIGHEST)


def _kmeans_body(s_ref, u_ref, pos_ref, h_ref, ids_ref, cen_ref):
    s = s_ref[...]
    u = u_ref[0]
    pos = pos_ref[...]
    ub = jnp.broadcast_to(u[:, None, :], (BB, T, D_USER))
    h = jnp.concatenate([s, ub], axis=2) + pos[None, :, :]
    h_ref[...] = h
    X = h.reshape(N, HIDDEN)
    XT = X.T

    kiota = lax.broadcasted_iota(jnp.int32, (BB, KP, N), 1)
    riota = lax.broadcasted_iota(jnp.int32, (BB, N), 0)
    bcol = lax.broadcasted_iota(jnp.int32, (BB, N), 1) // T
    giota = lax.broadcasted_iota(jnp.int32, (GK, N), 0)
    grow = lax.broadcasted_iota(jnp.int32, (GK, HIDDEN), 0)

    C0 = jnp.where((grow % KP) < K, h[:, :KP, :].reshape(GK, HIDDEN), PADC)

    def assign(C):
        G = lax.dot_general(C, XT, (((1,), (0,)), ((), ())), **_DOT)
        c2 = jnp.sum(C * C, axis=1, keepdims=True)
        d3 = (c2 - 2.0 * G).reshape(BB, KP, N)
        dmin = jnp.min(d3, axis=1)
        idsk = jnp.min(jnp.where(d3 == dmin[:, None, :], kiota, KP), axis=1)
        return jnp.sum(jnp.where(riota == bcol, idsk, 0), axis=0,
                       keepdims=True)

    def body(i, C):
        ids = assign(C)
        g = bcol[0:1, :] * KP + ids
        OT = (giota == g).astype(jnp.float32)
        sums = lax.dot_general(OT, X, (((1,), (0,)), ((), ())), **_DOT)
        cnt = jnp.sum(OT, axis=1, keepdims=True)
        return jnp.where(cnt > 0.0, sums / jnp.maximum(cnt, 1.0), C)

    C = lax.fori_loop(0, ITERS, body, C0)
    ids_ref[...] = assign(C)[None]
    cen_ref[...] = C.reshape(BB, KP, HIDDEN)


@jax.jit
def _kmeans_tc(s, u, pos):
    return pl.pallas_call(
        _kmeans_body,
        grid=(B // BB,),
        in_specs=[
            pl.BlockSpec((BB, T, D_ITEM), lambda i: (i, 0, 0)),
            pl.BlockSpec((1, BB, D_USER), lambda i: (i, 0, 0)),
            pl.BlockSpec((T, HIDDEN), lambda i: (0, 0)),
        ],
        out_specs=[
            pl.BlockSpec((BB, T, HIDDEN), lambda i: (i, 0, 0)),
            pl.BlockSpec((1, 1, N), lambda i: (i, 0, 0)),
            pl.BlockSpec((BB, KP, HIDDEN), lambda i: (i, 0, 0)),
        ],
        out_shape=[
            jax.ShapeDtypeStruct((B, T, HIDDEN), jnp.float32),
            jax.ShapeDtypeStruct((B // BB, 1, N), jnp.int32),
            jax.ShapeDtypeStruct((B, KP, HIDDEN), jnp.float32),
        ],
    )(s, u.reshape(B // BB, BB, D_USER), pos)


def kernel(user_ids, seq, pos_seqs, neg_seqs, item_table, user_table, pos_table):
    seq3d = seq.astype(jnp.int32).reshape(NW, ROWS_W // GCH, GCH)
    uids = user_ids.astype(jnp.int32)
    s3d, u = _sc_gather(seq3d, uids, item_table, user_table)
    s = s3d.reshape(B, T, D_ITEM)
    h, ids3, cen_r = _kmeans_tc(s, u, pos_table)
    return h, ids3.reshape(B, T), cen_r[:, :K, :]


# speedup vs baseline: 1.1308x; 1.0552x over previous
import functools

import jax
import jax.numpy as jnp
from jax import lax
from jax.experimental import pallas as pl
from jax.experimental.pallas import tpu as pltpu
from jax.experimental.pallas import tpu_sc as plsc

D_ITEM = 64
D_USER = 64
HIDDEN = 128
T = 200
B = 1024
K = 10
ITERS = 10

NW = 32
ROWS_W = (B * T) // NW
GCH = 128
NCH_G = 10
NSUP = ROWS_W // (GCH * NCH_G)
U_W = B // NW


def _sc_gather_body(seq_hbm, uid_hbm, item_hbm, user_hbm, s_out, u_out,
                    idx_v, rows_v, uidx_v, urows_v, sem):
    wid = lax.axis_index("s") * 2 + lax.axis_index("c")
    pltpu.sync_copy(uid_hbm.at[pl.ds(wid * U_W, U_W)], uidx_v)
    pltpu.async_copy(user_hbm.at[uidx_v], urows_v, sem).wait()
    pltpu.sync_copy(urows_v, u_out.at[pl.ds(wid * U_W, U_W)])
    pltpu.sync_copy(seq_hbm.at[wid], idx_v)

    def sup(s, _):
        for j in range(NCH_G):
            pltpu.async_copy(item_hbm.at[idx_v.at[s * NCH_G + j]],
                             rows_v.at[j], sem).wait()
        pltpu.sync_copy(
            rows_v,
            s_out.at[pl.ds(wid * (ROWS_W // GCH) + s * NCH_G, NCH_G)])
        return ()

    lax.fori_loop(0, NSUP, sup, (), unroll=False)


@jax.jit
def _sc_gather(seq3d, user_ids, item_table, user_table):
    mesh = plsc.VectorSubcoreMesh(core_axis_name="c", subcore_axis_name="s")
    f = pl.kernel(
        _sc_gather_body,
        mesh=mesh,
        out_type=[
            jax.ShapeDtypeStruct((B * T // GCH, GCH, D_ITEM), jnp.float32),
            jax.ShapeDtypeStruct((B, D_USER), jnp.float32),
        ],
        scratch_types=[
            pltpu.VMEM((ROWS_W // GCH, GCH), jnp.int32),
            pltpu.VMEM((NCH_G, GCH, D_ITEM), jnp.float32),
            pltpu.VMEM((U_W,), jnp.int32),
            pltpu.VMEM((U_W, D_USER), jnp.float32),
            pltpu.SemaphoreType.DMA,
        ],
        compiler_params=pltpu.CompilerParams(use_tc_tiling_on_sc=False),
    )
    return f(seq3d, user_ids, item_table, user_table)


BB = 8
KP = 16
GK = BB * KP
N = BB * T
PADC = 1e15
_DOT = dict(preferred_element_type=jnp.float32,
            precision=jax.lax.Precision.HIGHEST)


def _kmeans_body(s_ref, u_ref, pos_ref, h_ref, ids_ref, cen_ref):
    s = s_ref[...]
    u = u_ref[...]
    pos = pos_ref[...]
    ub = jnp.broadcast_to(u[:, None, :], (BB, T, D_USER))
    h = jnp.concatenate([s, ub], axis=2) + pos[None, :, :]
    h_ref[...] = h
    X = h.reshape(N, HIDDEN)
    XT = X.T

    kiota = lax.broadcasted_iota(jnp.int32, (BB, KP, N), 1)
    riota = lax.broadcasted_iota(jnp.int32, (BB, N), 0)
    bcol = lax.broadcasted_iota(jnp.int32, (BB, N), 1) // T
    giota = lax.broadcasted_iota(jnp.int32, (GK, N), 0)
    grow = lax.broadcasted_iota(jnp.int32, (GK, HIDDEN), 0)

    C0 = jnp.where((grow % KP) < K, h[:, :KP, :].reshape(GK, HIDDEN), PADC)

    def assign(C):
        Cm = -2.0 * C
        G = lax.dot_general(Cm, XT, (((1,), (0,)), ((), ())), **_DOT)
        c2 = 0.5 * jnp.sum(Cm * Cm, axis=1, keepdims=True)
        d3 = (0.5 * c2 + G).reshape(BB, KP, N)
        dmin = jnp.min(d3, axis=1)
        idsk = jnp.min(jnp.where(d3 == dmin[:, None, :], kiota, KP), axis=1)
        return jnp.sum(jnp.where(riota == bcol, idsk, 0), axis=0,
                       keepdims=True)

    def body(i, C):
        ids = assign(C)
        g = bcol[0:1, :] * KP + ids
        OT = (giota == g).astype(jnp.float32)
        sums = lax.dot_general(OT, X, (((1,), (0,)), ((), ())), **_DOT)
        cnt = jnp.sum(OT, axis=1, keepdims=True)
        return jnp.where(cnt > 0.0, sums / jnp.maximum(cnt, 1.0), C)

    C = lax.fori_loop(0, ITERS, body, C0)
    ids_ref[...] = assign(C)[None]
    cen_ref[...] = C.reshape(BB, KP, HIDDEN)


@jax.jit
def _kmeans_tc(s, u, pos):
    return pl.pallas_call(
        _kmeans_body,
        grid=(B // BB,),
        in_specs=[
            pl.BlockSpec((BB, T, D_ITEM), lambda i: (i, 0, 0)),
            pl.BlockSpec((BB, D_USER), lambda i: (i, 0)),
            pl.BlockSpec((T, HIDDEN), lambda i: (0, 0)),
        ],
        out_specs=[
            pl.BlockSpec((BB, T, HIDDEN), lambda i: (i, 0, 0)),
            pl.BlockSpec((1, 1, N), lambda i: (i, 0, 0)),
            pl.BlockSpec((BB, KP, HIDDEN), lambda i: (i, 0, 0)),
        ],
        out_shape=[
            jax.ShapeDtypeStruct((B, T, HIDDEN), jnp.float32),
            jax.ShapeDtypeStruct((B // BB, 1, N), jnp.int32),
            jax.ShapeDtypeStruct((B, KP, HIDDEN), jnp.float32),
        ],
    )(s, u, pos)


def kernel(user_ids, seq, pos_seqs, neg_seqs, item_table, user_table, pos_table):
    seq3d = seq.astype(jnp.int32).reshape(NW, ROWS_W // GCH, GCH)
    uids = user_ids.astype(jnp.int32)
    s3d, u = _sc_gather(seq3d, uids, item_table, user_table)
    s = s3d.reshape(B, T, D_ITEM)
    h, ids3, cen_r = _kmeans_tc(s, u, pos_table)
    return h, ids3.reshape(B, T), cen_r[:, :K, :]
